# baseline (device time: 94971 ns/iter reference)
import jax
import jax.numpy as jnp
from jax import lax
from jax.experimental import pallas as pl
from jax.experimental.pallas import tpu as pltpu

N_DEV = 32
N_CLS = N_DEV // 2
CHUNK = 1024 // N_DEV
G = 4


def kernel(x, w_mat):
    m, _ = x.shape
    _, n = w_mat.shape
    ng = n // G
    half = m // 2

    def body(x_ref, w_ref, out_ref, rs_buf, ps_buf, pr_buf,
             rs_ssem, rs_rsem, r3_ssem, r3_rsem,
             ag_ssem, ag_rsem, x2_ssem, x2_rsem):
        my = lax.axis_index("i")
        q = jnp.mod(my, 2)
        k = my // 2
        partner = my + 1 - 2 * q
        my_half = q * half
        my_row0 = my_half + k * CHUNK

        barrier_sem = pltpu.get_barrier_semaphore()
        pl.semaphore_signal(
            barrier_sem, inc=1, device_id=(partner,),
            device_id_type=pl.DeviceIdType.MESH,
        )
        for j in range(1, N_CLS):
            peer = jnp.mod(my + 2 * j, N_DEV)
            pl.semaphore_signal(
                barrier_sem, inc=1, device_id=(peer,),
                device_id_type=pl.DeviceIdType.MESH,
            )
        pl.semaphore_wait(barrier_sem, N_CLS)

        rs = [[None] * (2 * (N_CLS - 1)) for _ in range(G)]
        r3 = [None] * G
        ag = [[None] * (N_CLS - 1) for _ in range(G)]
        x2 = [None] * G

        def start_r2(g):
            c0 = g * ng
            out_ref[:, pl.ds(c0, ng)] = jnp.dot(
                x_ref[:, :], w_ref[:, pl.ds(c0, ng)],
                preferred_element_type=jnp.float32,
            )
            for j in range(1, N_CLS):
                tgt = jnp.mod(my + 2 * j, N_DEV)
                tk = tgt // 2
                for b in range(2):
                    rdma = pltpu.make_async_remote_copy(
                        src_ref=out_ref.at[pl.ds(b * half + tk * CHUNK, CHUNK),
                                           pl.ds(c0, ng)],
                        dst_ref=rs_buf.at[g, j - 1, b],
                        send_sem=rs_ssem.at[g, j - 1, b],
                        recv_sem=rs_rsem.at[g, j - 1, b],
                        device_id=(tgt,),
                        device_id_type=pl.DeviceIdType.MESH,
                    )
                    rdma.start()
                    rs[g][2 * (j - 1) + b] = rdma

        def start_r3(g):
            c0 = g * ng
            for r in rs[g]:
                r.wait_recv()
            sum0 = out_ref[pl.ds(k * CHUNK, CHUNK), pl.ds(c0, ng)] + jnp.sum(
                rs_buf[g, :, 0], axis=0
            )
            sum1 = out_ref[pl.ds(half + k * CHUNK, CHUNK),
                           pl.ds(c0, ng)] + jnp.sum(rs_buf[g, :, 1], axis=0)
            even = (q == 0)
            my_sum = jnp.where(even, sum0, sum1)
            partner_sum = jnp.where(even, sum1, sum0)
            out_ref[pl.ds(my_row0, CHUNK), pl.ds(c0, ng)] = my_sum
            ps_buf[g] = partner_sum
            rdma = pltpu.make_async_remote_copy(
                src_ref=ps_buf.at[g],
                dst_ref=pr_buf.at[g],
                send_sem=r3_ssem.at[g],
                recv_sem=r3_rsem.at[g],
                device_id=(partner,),
                device_id_type=pl.DeviceIdType.MESH,
            )
            rdma.start()
            r3[g] = rdma

        def start_ag1(g):
            c0 = g * ng
            r3[g].wait_recv()
            out_ref[pl.ds(my_row0, CHUNK), pl.ds(c0, ng)] = (
                out_ref[pl.ds(my_row0, CHUNK), pl.ds(c0, ng)] + pr_buf[g]
            )
            for j in range(1, N_CLS):
                tgt = jnp.mod(my + 2 * j, N_DEV)
                rdma = pltpu.make_async_remote_copy(
                    src_ref=out_ref.at[pl.ds(my_row0, CHUNK), pl.ds(c0, ng)],
                    dst_ref=out_ref.at[pl.ds(my_row0, CHUNK), pl.ds(c0, ng)],
                    send_sem=ag_ssem.at[g, j - 1],
                    recv_sem=ag_rsem.at[g, j - 1],
                    device_id=(tgt,),
                    device_id_type=pl.DeviceIdType.MESH,
                )
                rdma.start()
                ag[g][j - 1] = rdma

        def start_x2(g):
            c0 = g * ng
            for r in ag[g]:
                r.wait_recv()
            rdma = pltpu.make_async_remote_copy(
                src_ref=out_ref.at[pl.ds(my_half, half), pl.ds(c0, ng)],
                dst_ref=out_ref.at[pl.ds(my_half, half), pl.ds(c0, ng)],
                send_sem=x2_ssem.at[g],
                recv_sem=x2_rsem.at[g],
                device_id=(partner,),
                device_id_type=pl.DeviceIdType.MESH,
            )
            rdma.start()
            x2[g] = rdma

        stages = [start_r2, start_r3, start_ag1, start_x2]
        for step in range(G + len(stages) - 1):
            for s, fn in enumerate(stages):
                g = step - s
                if 0 <= g < G:
                    fn(g)

        for g in range(G):
            x2[g].wait_recv()
            r3[g].wait_send()
            x2[g].wait_send()
            for r in rs[g]:
                r.wait_send()
            for r in ag[g]:
                r.wait_send()

    return pl.pallas_call(
        body,
        out_shape=jax.ShapeDtypeStruct((m, n), jnp.float32),
        in_specs=[
            pl.BlockSpec(memory_space=pltpu.VMEM),
            pl.BlockSpec(memory_space=pltpu.VMEM),
        ],
        out_specs=pl.BlockSpec(memory_space=pltpu.VMEM),
        scratch_shapes=[
            pltpu.VMEM((G, N_CLS - 1, 2, CHUNK, ng), jnp.float32),
            pltpu.VMEM((G, CHUNK, ng), jnp.float32),
            pltpu.VMEM((G, CHUNK, ng), jnp.float32),
            pltpu.SemaphoreType.DMA((G, N_CLS - 1, 2)),
            pltpu.SemaphoreType.DMA((G, N_CLS - 1, 2)),
            pltpu.SemaphoreType.DMA((G,)),
            pltpu.SemaphoreType.DMA((G,)),
            pltpu.SemaphoreType.DMA((G, N_CLS - 1)),
            pltpu.SemaphoreType.DMA((G, N_CLS - 1)),
            pltpu.SemaphoreType.DMA((G,)),
            pltpu.SemaphoreType.DMA((G,)),
        ],
        compiler_params=pltpu.CompilerParams(collective_id=0),
    )(x, w_mat)


# device time: 51300 ns/iter; 1.8513x vs baseline; 1.8513x over previous
import jax
import jax.numpy as jnp
from jax import lax
from jax.experimental import pallas as pl
from jax.experimental.pallas import tpu as pltpu

N_DEV = 32
N_CLS = N_DEV // 2
CHUNK = 1024 // N_DEV
G = 4
WIRE = jnp.bfloat16


def kernel(x, w_mat):
    m, _ = x.shape
    _, n = w_mat.shape
    ng = n // G
    half = m // 2

    def body(x_ref, w_ref, out_ref,
             sb_buf, rx_buf, cb_buf, rs_buf, ab_buf, agb_buf, xb_buf,
             xr_buf,
             r1_ssem, r1_rsem, rs_ssem, rs_rsem,
             ag_ssem, ag_rsem, x2_ssem, x2_rsem):
        my = lax.axis_index("i")
        q = jnp.mod(my, 2)
        partner = my + 1 - 2 * q
        my_half = q * half
        other_half = (1 - q) * half
        my_row0 = my_half + (my // 2) * CHUNK

        barrier_sem = pltpu.get_barrier_semaphore()
        pl.semaphore_signal(
            barrier_sem, inc=1, device_id=(partner,),
            device_id_type=pl.DeviceIdType.MESH,
        )
        for j in range(1, N_CLS):
            peer = jnp.mod(my + 2 * j, N_DEV)
            pl.semaphore_signal(
                barrier_sem, inc=1, device_id=(peer,),
                device_id_type=pl.DeviceIdType.MESH,
            )
        pl.semaphore_wait(barrier_sem, N_CLS)

        r1 = [None] * G
        rs = [[None] * (N_CLS - 1) for _ in range(G)]
        ag = [[None] * (N_CLS - 1) for _ in range(G)]
        x2 = [None] * G

        def start_r1(g):
            c0 = g * ng
            out_ref[:, pl.ds(c0, ng)] = jnp.dot(
                x_ref[:, :], w_ref[:, pl.ds(c0, ng)],
                preferred_element_type=jnp.float32,
            )
            sb_buf[g] = out_ref[pl.ds(other_half, half),
                                pl.ds(c0, ng)].astype(WIRE)
            rdma = pltpu.make_async_remote_copy(
                src_ref=sb_buf.at[g],
                dst_ref=rx_buf.at[g],
                send_sem=r1_ssem.at[g],
                recv_sem=r1_rsem.at[g],
                device_id=(partner,),
                device_id_type=pl.DeviceIdType.MESH,
            )
            rdma.start()
            r1[g] = rdma

        def start_r2(g):
            c0 = g * ng
            r1[g].wait_recv()
            combined = out_ref[pl.ds(my_half, half),
                               pl.ds(c0, ng)] + rx_buf[g].astype(jnp.float32)
            out_ref[pl.ds(my_half, half), pl.ds(c0, ng)] = combined
            cb_buf[g] = combined.astype(WIRE)
            for j in range(1, N_CLS):
                tgt = jnp.mod(my + 2 * j, N_DEV)
                rdma = pltpu.make_async_remote_copy(
                    src_ref=cb_buf.at[g, pl.ds((tgt // 2) * CHUNK, CHUNK)],
                    dst_ref=rs_buf.at[g, j - 1],
                    send_sem=rs_ssem.at[g, j - 1],
                    recv_sem=rs_rsem.at[g, j - 1],
                    device_id=(tgt,),
                    device_id_type=pl.DeviceIdType.MESH,
                )
                rdma.start()
                rs[g][j - 1] = rdma

        def start_ag1(g):
            c0 = g * ng
            for r in rs[g]:
                r.wait_recv()
            acc = out_ref[pl.ds(my_row0, CHUNK), pl.ds(c0, ng)] + jnp.sum(
                rs_buf[g].astype(jnp.float32), axis=0
            )
            out_ref[pl.ds(my_row0, CHUNK), pl.ds(c0, ng)] = acc
            ab_buf[g] = acc.astype(WIRE)
            for j in range(1, N_CLS):
                tgt = jnp.mod(my + 2 * j, N_DEV)
                rdma = pltpu.make_async_remote_copy(
                    src_ref=ab_buf.at[g],
                    dst_ref=agb_buf.at[g, j - 1],
                    send_sem=ag_ssem.at[g, j - 1],
                    recv_sem=ag_rsem.at[g, j - 1],
                    device_id=(tgt,),
                    device_id_type=pl.DeviceIdType.MESH,
                )
                rdma.start()
                ag[g][j - 1] = rdma

        def start_x2(g):
            c0 = g * ng
            for r in ag[g]:
                r.wait_recv()
            for j in range(1, N_CLS):
                src_dev = jnp.mod(my - 2 * j, N_DEV)
                row0 = my_half + (src_dev // 2) * CHUNK
                out_ref[pl.ds(row0, CHUNK), pl.ds(c0, ng)] = (
                    agb_buf[g, j - 1].astype(jnp.float32)
                )
            xb_buf[g] = out_ref[pl.ds(my_half, half),
                                pl.ds(c0, ng)].astype(WIRE)
            rdma = pltpu.make_async_remote_copy(
                src_ref=xb_buf.at[g],
                dst_ref=xr_buf.at[g],
                send_sem=x2_ssem.at[g],
                recv_sem=x2_rsem.at[g],
                device_id=(partner,),
                device_id_type=pl.DeviceIdType.MESH,
            )
            rdma.start()
            x2[g] = rdma

        stages = [start_r1, start_r2, start_ag1, start_x2]
        for step in range(G + len(stages) - 1):
            for s, fn in enumerate(stages):
                g = step - s
                if 0 <= g < G:
                    fn(g)

        for g in range(G):
            c0 = g * ng
            x2[g].wait_recv()
            out_ref[pl.ds(other_half, half), pl.ds(c0, ng)] = (
                xr_buf[g].astype(jnp.float32)
            )
            r1[g].wait_send()
            x2[g].wait_send()
            for r in rs[g]:
                r.wait_send()
            for r in ag[g]:
                r.wait_send()

    return pl.pallas_call(
        body,
        out_shape=jax.ShapeDtypeStruct((m, n), jnp.float32),
        in_specs=[
            pl.BlockSpec(memory_space=pltpu.VMEM),
            pl.BlockSpec(memory_space=pltpu.VMEM),
        ],
        out_specs=pl.BlockSpec(memory_space=pltpu.VMEM),
        scratch_shapes=[
            pltpu.VMEM((G, half, ng), WIRE),
            pltpu.VMEM((G, half, ng), WIRE),
            pltpu.VMEM((G, half, ng), WIRE),
            pltpu.VMEM((G, N_CLS - 1, CHUNK, ng), WIRE),
            pltpu.VMEM((G, CHUNK, ng), WIRE),
            pltpu.VMEM((G, N_CLS - 1, CHUNK, ng), WIRE),
            pltpu.VMEM((G, half, ng), WIRE),
            pltpu.VMEM((G, half, ng), WIRE),
            pltpu.SemaphoreType.DMA((G,)),
            pltpu.SemaphoreType.DMA((G,)),
            pltpu.SemaphoreType.DMA((G, N_CLS - 1)),
            pltpu.SemaphoreType.DMA((G, N_CLS - 1)),
            pltpu.SemaphoreType.DMA((G, N_CLS - 1)),
            pltpu.SemaphoreType.DMA((G, N_CLS - 1)),
            pltpu.SemaphoreType.DMA((G,)),
            pltpu.SemaphoreType.DMA((G,)),
        ],
        compiler_params=pltpu.CompilerParams(collective_id=0),
    )(x, w_mat)


# device time: 50224 ns/iter; 1.8909x vs baseline; 1.0214x over previous
import jax
import jax.numpy as jnp
from jax import lax
from jax.experimental import pallas as pl
from jax.experimental.pallas import tpu as pltpu

N_DEV = 32
N_CLS = N_DEV // 2
CHUNK = 1024 // N_DEV
G = 4
WIRE = jnp.bfloat16


def kernel(x, w_mat):
    m, kk = x.shape
    _, n = w_mat.shape
    ng = n // G
    half = m // 2

    def body(x_ref, w_ref, out_ref,
             xs_buf, ws_buf, xp_buf, wp_buf,
             cb_buf, rs_buf, ab_buf, agb_buf, xb_buf, xr_buf,
             xw_ssem, xw_rsem, rs_ssem, rs_rsem,
             ag_ssem, ag_rsem, x2_ssem, x2_rsem):
        my = lax.axis_index("i")
        q = jnp.mod(my, 2)
        partner = my + 1 - 2 * q
        my_half = q * half
        other_half = (1 - q) * half
        my_row0 = my_half + (my // 2) * CHUNK

        barrier_sem = pltpu.get_barrier_semaphore()
        pl.semaphore_signal(
            barrier_sem, inc=1, device_id=(partner,),
            device_id_type=pl.DeviceIdType.MESH,
        )
        for j in range(1, N_CLS):
            peer = jnp.mod(my + 2 * j, N_DEV)
            pl.semaphore_signal(
                barrier_sem, inc=1, device_id=(peer,),
                device_id_type=pl.DeviceIdType.MESH,
            )
        pl.semaphore_wait(barrier_sem, N_CLS)

        xs_buf[:, :] = x_ref[pl.ds(other_half, half), :].astype(WIRE)
        ws_buf[:, :] = w_ref[:, :].astype(WIRE)
        xw = []
        for sl, (src, dst) in enumerate(((xs_buf, xp_buf), (ws_buf, wp_buf))):
            rdma = pltpu.make_async_remote_copy(
                src_ref=src,
                dst_ref=dst,
                send_sem=xw_ssem.at[sl],
                recv_sem=xw_rsem.at[sl],
                device_id=(partner,),
                device_id_type=pl.DeviceIdType.MESH,
            )
            rdma.start()
            xw.append(rdma)

        rs = [[None] * (N_CLS - 1) for _ in range(G)]
        ag = [[None] * (N_CLS - 1) for _ in range(G)]
        x2 = [None] * G

        def start_r2(g):
            c0 = g * ng
            if g == 0:
                for r in xw:
                    r.wait_recv()
            combined = jnp.dot(
                x_ref[pl.ds(my_half, half), :], w_ref[:, pl.ds(c0, ng)],
                preferred_element_type=jnp.float32,
            ) + jnp.dot(
                xp_buf[:, :], wp_buf[:, pl.ds(c0, ng)],
                preferred_element_type=jnp.float32,
            )
            out_ref[pl.ds(my_half, half), pl.ds(c0, ng)] = combined
            cb_buf[g] = combined.astype(WIRE)
            for j in range(1, N_CLS):
                tgt = jnp.mod(my + 2 * j, N_DEV)
                rdma = pltpu.make_async_remote_copy(
                    src_ref=cb_buf.at[g, pl.ds((tgt // 2) * CHUNK, CHUNK)],
                    dst_ref=rs_buf.at[g, j - 1],
                    send_sem=rs_ssem.at[g, j - 1],
                    recv_sem=rs_rsem.at[g, j - 1],
                    device_id=(tgt,),
                    device_id_type=pl.DeviceIdType.MESH,
                )
                rdma.start()
                rs[g][j - 1] = rdma

        def start_ag1(g):
            c0 = g * ng
            for r in rs[g]:
                r.wait_recv()
            acc = out_ref[pl.ds(my_row0, CHUNK), pl.ds(c0, ng)] + jnp.sum(
                rs_buf[g].astype(jnp.float32), axis=0
            )
            out_ref[pl.ds(my_row0, CHUNK), pl.ds(c0, ng)] = acc
            ab_buf[g] = acc.astype(WIRE)
            for j in range(1, N_CLS):
                tgt = jnp.mod(my + 2 * j, N_DEV)
                rdma = pltpu.make_async_remote_copy(
                    src_ref=ab_buf.at[g],
                    dst_ref=agb_buf.at[g, j - 1],
                    send_sem=ag_ssem.at[g, j - 1],
                    recv_sem=ag_rsem.at[g, j - 1],
                    device_id=(tgt,),
                    device_id_type=pl.DeviceIdType.MESH,
                )
                rdma.start()
                ag[g][j - 1] = rdma

        def start_x2(g):
            c0 = g * ng
            for r in ag[g]:
                r.wait_recv()
            for j in range(1, N_CLS):
                src_dev = jnp.mod(my - 2 * j, N_DEV)
                row0 = my_half + (src_dev // 2) * CHUNK
                out_ref[pl.ds(row0, CHUNK), pl.ds(c0, ng)] = (
                    agb_buf[g, j - 1].astype(jnp.float32)
                )
            xb_buf[g] = out_ref[pl.ds(my_half, half),
                                pl.ds(c0, ng)].astype(WIRE)
            rdma = pltpu.make_async_remote_copy(
                src_ref=xb_buf.at[g],
                dst_ref=xr_buf.at[g],
                send_sem=x2_ssem.at[g],
                recv_sem=x2_rsem.at[g],
                device_id=(partner,),
                device_id_type=pl.DeviceIdType.MESH,
            )
            rdma.start()
            x2[g] = rdma

        stages = [start_r2, start_ag1, start_x2]
        for step in range(G + len(stages) - 1):
            for s, fn in enumerate(stages):
                g = step - s
                if 0 <= g < G:
                    fn(g)

        for g in range(G):
            c0 = g * ng
            x2[g].wait_recv()
            out_ref[pl.ds(other_half, half), pl.ds(c0, ng)] = (
                xr_buf[g].astype(jnp.float32)
            )
            x2[g].wait_send()
            for r in rs[g]:
                r.wait_send()
            for r in ag[g]:
                r.wait_send()
        for r in xw:
            r.wait_send()

    return pl.pallas_call(
        body,
        out_shape=jax.ShapeDtypeStruct((m, n), jnp.float32),
        in_specs=[
            pl.BlockSpec(memory_space=pltpu.VMEM),
            pl.BlockSpec(memory_space=pltpu.VMEM),
        ],
        out_specs=pl.BlockSpec(memory_space=pltpu.VMEM),
        scratch_shapes=[
            pltpu.VMEM((half, kk), WIRE),
            pltpu.VMEM((kk, n), WIRE),
            pltpu.VMEM((half, kk), WIRE),
            pltpu.VMEM((kk, n), WIRE),
            pltpu.VMEM((G, half, ng), WIRE),
            pltpu.VMEM((G, N_CLS - 1, CHUNK, ng), WIRE),
            pltpu.VMEM((G, CHUNK, ng), WIRE),
            pltpu.VMEM((G, N_CLS - 1, CHUNK, ng), WIRE),
            pltpu.VMEM((G, half, ng), WIRE),
            pltpu.VMEM((G, half, ng), WIRE),
            pltpu.SemaphoreType.DMA((2,)),
            pltpu.SemaphoreType.DMA((2,)),
            pltpu.SemaphoreType.DMA((G, N_CLS - 1)),
            pltpu.SemaphoreType.DMA((G, N_CLS - 1)),
            pltpu.SemaphoreType.DMA((G, N_CLS - 1)),
            pltpu.SemaphoreType.DMA((G, N_CLS - 1)),
            pltpu.SemaphoreType.DMA((G,)),
            pltpu.SemaphoreType.DMA((G,)),
        ],
        compiler_params=pltpu.CompilerParams(collective_id=0),
    )(x, w_mat)
